# baseline (speedup 1.0000x reference)
import functools
import math

import jax
import jax.numpy as jnp
from jax import lax
from jax.experimental import pallas as pl
from jax.experimental.pallas import tpu as pltpu
from jax.experimental.pallas import tpu_sc as plsc

_SCALING = 32.0
_MARGIN = 0.5
_COS_M = math.cos(_MARGIN)
_SIN_M = math.sin(_MARGIN)
_TH = math.cos(math.pi - _MARGIN)
_MM = math.sin(math.pi - _MARGIN) * _MARGIN

_B = 1024
_C = 100000
_CBLK = 2048
_NBLK = (_C + _CBLK - 1) // _CBLK

_NC = 2
_NS = 16
_NW = _NC * _NS
_EPW = _B // _NW


def _stream_body(t_ref, x_ref, out_ref, s_ref, g_ref):
    j = pl.program_id(0)
    x = x_ref[...]
    v = jnp.clip(x, -1.0, 1.0) * _SCALING
    out_ref[...] = v
    row = lax.broadcasted_iota(jnp.int32, (_CBLK, _B), 0) + j * _CBLK
    is_t = row == t_ref[...]
    e = jnp.where(row < _C, jnp.exp(v - _SCALING), 0.0)

    @pl.when(j == 0)
    def _():
        s_ref[...] = jnp.zeros_like(s_ref)
        g_ref[...] = jnp.zeros_like(g_ref)

    s_ref[...] += jnp.sum(e, axis=0, keepdims=True)
    g_ref[...] += jnp.sum(jnp.where(is_t, v, 0.0), axis=0, keepdims=True)


def _loss_body(s_ref, g_ref, loss_ref, phi_ref):
    s = s_ref[...]
    g = g_ref[...]
    c = g * (1.0 / _SCALING)
    sine = jnp.sqrt(jnp.maximum(1.0 - c * c, 1e-7))
    phi = c * _COS_M - sine * _SIN_M
    phi = jnp.where(c - _TH > 0, phi, c - _MM)
    outt = phi * _SCALING
    lse = _SCALING + jnp.log(s - jnp.exp(g - _SCALING) + jnp.exp(outt - _SCALING))
    loss_ref[...] = jnp.sum(lse - outt, axis=(0, 1), keepdims=True) * (1.0 / _B)
    phi_ref[...] = outt


@functools.cache
def _make_scatter_kernel():
    mesh = plsc.VectorSubcoreMesh(
        core_axis_name="c", subcore_axis_name="s", num_cores=_NC, num_subcores=_NS
    )

    @functools.partial(
        pl.kernel,
        mesh=mesh,
        scratch_types=[
            pltpu.VMEM((_EPW,), jnp.int32),
            pltpu.VMEM((_EPW,), jnp.int32),
            pltpu.VMEM((_EPW,), jnp.float32),
            pltpu.SemaphoreType.DMA,
        ],
    )
    def _scatter_kernel(t_hbm, val_hbm, out_hbm, t_v, idx_v, val_v, sem):
        wid = lax.axis_index("s") * _NC + lax.axis_index("c")
        base = wid * _EPW
        pltpu.sync_copy(t_hbm.at[pl.ds(base, _EPW)], t_v)
        pltpu.sync_copy(val_hbm.at[pl.ds(base, _EPW)], val_v)
        for k in range(_EPW // 16):
            t16 = t_v[pl.ds(k * 16, 16)]
            b16 = base + k * 16 + lax.iota(jnp.int32, 16)
            idx_v[pl.ds(k * 16, 16)] = t16 * _B + b16
        pltpu.async_copy(val_v, out_hbm.at[idx_v], sem).wait()

    return _scatter_kernel


def kernel(cosine_fea2cen, targets):
    xt = cosine_fea2cen.T
    t2 = targets.reshape(1, _B)
    outt, s, g = pl.pallas_call(
        _stream_body,
        grid=(_NBLK,),
        in_specs=[
            pl.BlockSpec((1, _B), lambda j: (0, 0)),
            pl.BlockSpec((_CBLK, _B), lambda j: (j, 0)),
        ],
        out_specs=[
            pl.BlockSpec((_CBLK, _B), lambda j: (j, 0)),
            pl.BlockSpec((1, _B), lambda j: (0, 0)),
            pl.BlockSpec((1, _B), lambda j: (0, 0)),
        ],
        out_shape=[
            jax.ShapeDtypeStruct((_C, _B), jnp.float32),
            jax.ShapeDtypeStruct((1, _B), jnp.float32),
            jax.ShapeDtypeStruct((1, _B), jnp.float32),
        ],
    )(t2, xt)

    loss, phi32 = pl.pallas_call(
        _loss_body,
        out_shape=[
            jax.ShapeDtypeStruct((1, 1), jnp.float32),
            jax.ShapeDtypeStruct((1, _B), jnp.float32),
        ],
    )(s, g)

    out_ref = jax.new_ref(outt.reshape(_C * _B))
    _make_scatter_kernel()(targets, phi32.reshape(_B), out_ref)
    out_final = out_ref[...].reshape(_C, _B).T
    return (loss[0, 0], out_final)


# speedup vs baseline: 3.4802x; 3.4802x over previous
import functools
import math

import jax
import jax.numpy as jnp
from jax import lax
from jax.experimental import pallas as pl
from jax.experimental.pallas import tpu as pltpu
from jax.experimental.pallas import tpu_sc as plsc

_SCALING = 32.0
_MARGIN = 0.5
_COS_M = math.cos(_MARGIN)
_SIN_M = math.sin(_MARGIN)
_TH = math.cos(math.pi - _MARGIN)
_MM = math.sin(math.pi - _MARGIN) * _MARGIN

_B = 1024
_C = 100000
_CBLK = 2048
_NBLK = (_C + _CBLK - 1) // _CBLK

_NROWS = (_C // 8) * (_B // 128) * 8

_NC = 2
_NS = 16
_NW = _NC * _NS
_EPW = _B // _NW


@functools.cache
def _make_gather_kernel():
    mesh = plsc.VectorSubcoreMesh(
        core_axis_name="c", subcore_axis_name="s", num_cores=_NC, num_subcores=_NS
    )

    @functools.partial(
        pl.kernel,
        mesh=mesh,
        out_type=jax.ShapeDtypeStruct((_B,), jnp.float32),
        scratch_types=[
            pltpu.VMEM((_EPW,), jnp.int32),
            pltpu.VMEM((_EPW,), jnp.int32),
            pltpu.VMEM((_EPW,), jnp.float32),
            pltpu.SemaphoreType.DMA,
        ],
    )
    def _gather_kernel(x1d_hbm, t_hbm, ct_hbm, t_v, idx_v, val_v, sem):
        wid = lax.axis_index("s") * _NC + lax.axis_index("c")
        base = wid * _EPW
        pltpu.sync_copy(t_hbm.at[pl.ds(base, _EPW)], t_v)
        for k in range(_EPW // 16):
            t16 = t_v[pl.ds(k * 16, 16)]
            b16 = base + k * 16 + lax.iota(jnp.int32, 16)
            r16 = ((t16 >> 3) << 6) + ((b16 >> 7) << 3) + (t16 & 7)
            idx_v[pl.ds(k * 16, 16)] = (r16 << 7) + (b16 & 127)
        pltpu.async_copy(x1d_hbm.at[idx_v], val_v, sem).wait()
        pltpu.sync_copy(val_v, ct_hbm.at[pl.ds(base, _EPW)])

    return _gather_kernel


def _phi_body(ct_ref, phi_ref):
    c = jnp.clip(ct_ref[...], -1.0, 1.0)
    sine = jnp.sqrt(jnp.maximum(1.0 - c * c, 1e-7))
    phi = c * _COS_M - sine * _SIN_M
    phi = jnp.where(c - _TH > 0, phi, c - _MM)
    phi_ref[...] = phi * _SCALING


def _stream_body(t_ref, phi_ref, x_ref, out_ref, loss_ref, s_acc):
    j = pl.program_id(0)
    x = x_ref[...]
    v = jnp.clip(x, -1.0, 1.0) * _SCALING
    row = lax.broadcasted_iota(jnp.int32, (_CBLK, _B), 0) + j * _CBLK
    is_t = row == t_ref[...]
    out = jnp.where(is_t, phi_ref[...], v)
    out_ref[...] = out
    e = jnp.where(row < _C, jnp.exp(out - _SCALING), 0.0)

    @pl.when(j == 0)
    def _():
        s_acc[...] = jnp.zeros_like(s_acc)

    s_acc[...] += jnp.sum(e, axis=0, keepdims=True)

    @pl.when(j == _NBLK - 1)
    def _():
        nll = _SCALING + jnp.log(s_acc[...]) - phi_ref[...]
        loss_ref[...] = jnp.sum(nll, axis=(0, 1), keepdims=True) * (1.0 / _B)


def kernel(cosine_fea2cen, targets):
    xt = cosine_fea2cen.T
    x1d = (
        xt.reshape(_C // 8, 8, _B // 128, 128)
        .transpose(0, 2, 1, 3)
        .reshape(_NROWS * 128)
    )
    ct = _make_gather_kernel()(x1d, targets)

    phi32 = pl.pallas_call(
        _phi_body,
        out_shape=jax.ShapeDtypeStruct((1, _B), jnp.float32),
    )(ct.reshape(1, _B))

    t2 = targets.reshape(1, _B)
    outt, loss = pl.pallas_call(
        _stream_body,
        grid=(_NBLK,),
        in_specs=[
            pl.BlockSpec((1, _B), lambda j: (0, 0)),
            pl.BlockSpec((1, _B), lambda j: (0, 0)),
            pl.BlockSpec((_CBLK, _B), lambda j: (j, 0)),
        ],
        out_specs=[
            pl.BlockSpec((_CBLK, _B), lambda j: (j, 0)),
            pl.BlockSpec((1, 1), lambda j: (0, 0)),
        ],
        out_shape=[
            jax.ShapeDtypeStruct((_C, _B), jnp.float32),
            jax.ShapeDtypeStruct((1, 1), jnp.float32),
        ],
        scratch_shapes=[
            pltpu.VMEM((1, _B), jnp.float32),
        ],
    )(t2, phi32, xt)
    return (loss[0, 0], outt.T)


# speedup vs baseline: 3.5012x; 1.0061x over previous
import functools
import math

import jax
import jax.numpy as jnp
from jax import lax
from jax.experimental import pallas as pl
from jax.experimental.pallas import tpu as pltpu
from jax.experimental.pallas import tpu_sc as plsc

_SCALING = 32.0
_MARGIN = 0.5
_COS_M = math.cos(_MARGIN)
_SIN_M = math.sin(_MARGIN)
_TH = math.cos(math.pi - _MARGIN)
_MM = math.sin(math.pi - _MARGIN) * _MARGIN

_B = 1024
_C = 100000
_CBLK = 2048
_NBLK = (_C + _CBLK - 1) // _CBLK

_NROWS = (_C // 8) * (_B // 128) * 8

_NC = 2
_NS = 16
_NW = _NC * _NS
_EPW = _B // _NW


@functools.cache
def _make_gather_kernel():
    mesh = plsc.VectorSubcoreMesh(
        core_axis_name="c", subcore_axis_name="s", num_cores=_NC, num_subcores=_NS
    )

    @functools.partial(
        pl.kernel,
        mesh=mesh,
        out_type=jax.ShapeDtypeStruct((_B,), jnp.float32),
        scratch_types=[
            pltpu.VMEM((_EPW,), jnp.int32),
            pltpu.VMEM((_EPW,), jnp.int32),
            pltpu.VMEM((_EPW,), jnp.float32),
            pltpu.SemaphoreType.DMA,
        ],
    )
    def _gather_kernel(x1d_hbm, t_hbm, ct_hbm, t_v, idx_v, val_v, sem):
        wid = lax.axis_index("s") * _NC + lax.axis_index("c")
        base = wid * _EPW
        pltpu.sync_copy(t_hbm.at[pl.ds(base, _EPW)], t_v)
        for k in range(_EPW // 16):
            t16 = t_v[pl.ds(k * 16, 16)]
            b16 = base + k * 16 + lax.iota(jnp.int32, 16)
            r16 = ((t16 >> 3) << 6) + ((b16 >> 7) << 3) + (t16 & 7)
            idx_v[pl.ds(k * 16, 16)] = (r16 << 7) + (b16 & 127)
        pltpu.async_copy(x1d_hbm.at[idx_v], val_v, sem).wait()
        pltpu.sync_copy(val_v, ct_hbm.at[pl.ds(base, _EPW)])

    return _gather_kernel


def _stream_body(t_ref, ct_ref, x_ref, out_ref, loss_ref, s_acc, phi_v):
    j = pl.program_id(0)

    @pl.when(j == 0)
    def _():
        c = jnp.clip(ct_ref[...], -1.0, 1.0)
        sine = jnp.sqrt(jnp.maximum(1.0 - c * c, 1e-7))
        phi = c * _COS_M - sine * _SIN_M
        phi = jnp.where(c - _TH > 0, phi, c - _MM)
        phi_v[...] = phi * _SCALING
        s_acc[...] = jnp.zeros_like(s_acc)

    x = x_ref[...]
    v = jnp.clip(x, -1.0, 1.0) * _SCALING
    row = lax.broadcasted_iota(jnp.int32, (_CBLK, _B), 0) + j * _CBLK
    is_t = row == t_ref[...]
    out = jnp.where(is_t, phi_v[...], v)
    out_ref[...] = out
    e = jnp.where(row < _C, jnp.exp(out - _SCALING), 0.0)
    s_acc[...] += jnp.sum(e, axis=0, keepdims=True)

    @pl.when(j == _NBLK - 1)
    def _():
        nll = _SCALING + jnp.log(s_acc[...]) - phi_v[...]
        loss_ref[...] = jnp.sum(nll, axis=(0, 1), keepdims=True) * (1.0 / _B)


def kernel(cosine_fea2cen, targets):
    xt = cosine_fea2cen.T
    x1d = (
        xt.reshape(_C // 8, 8, _B // 128, 128)
        .transpose(0, 2, 1, 3)
        .reshape(_NROWS * 128)
    )
    ct = _make_gather_kernel()(x1d, targets)

    t2 = targets.reshape(1, _B)
    outt, loss = pl.pallas_call(
        _stream_body,
        grid=(_NBLK,),
        in_specs=[
            pl.BlockSpec((1, _B), lambda j: (0, 0)),
            pl.BlockSpec((1, _B), lambda j: (0, 0)),
            pl.BlockSpec((_CBLK, _B), lambda j: (j, 0)),
        ],
        out_specs=[
            pl.BlockSpec((_CBLK, _B), lambda j: (j, 0)),
            pl.BlockSpec((1, 1), lambda j: (0, 0)),
        ],
        out_shape=[
            jax.ShapeDtypeStruct((_C, _B), jnp.float32),
            jax.ShapeDtypeStruct((1, 1), jnp.float32),
        ],
        scratch_shapes=[
            pltpu.VMEM((1, _B), jnp.float32),
            pltpu.VMEM((1, _B), jnp.float32),
        ],
    )(t2, ct.reshape(1, _B), xt)
    return (loss[0, 0], outt.T)
